# baseline (speedup 1.0000x reference)
import functools

import jax
import jax.numpy as jnp
from jax import lax
from jax.experimental import pallas as pl
from jax.experimental.pallas import tpu as pltpu
from jax.experimental.pallas import tpu_sc as plsc

B = 4
C = 16
V = 64
M = V * V * V
N = 100000

NUM_TILES = 16
PT = 6272
NP = NUM_TILES * PT
SUB = 448
NSUB = PT // SUB
BLOCKS_PER_SUB = SUB // 16

RING_CHUNKS = 4
RING_ROWS = RING_CHUNKS * 128
FCH = 512

ACC_ROWS = 16 * V * V
SHARE = ACC_ROWS // NUM_TILES

ZCHUNK = 128


def _corner_terms(px, py, pz, pid_ok, lo, nrows):
    fx = (px + 0.5) * V
    fy = (py + 0.5) * V
    fz = (pz + 0.5) * V
    x0 = fx.astype(jnp.int32)
    y0 = fy.astype(jnp.int32)
    z0 = fz.astype(jnp.int32)
    rx = fx - x0.astype(jnp.float32)
    ry = fy - y0.astype(jnp.float32)
    rz = fz - z0.astype(jnp.float32)
    x1 = x0 + 1
    y1 = y0 + 1
    z1 = z0 + 1
    wx1 = jnp.where(x1 < V, rx, 0.0)
    wy1 = jnp.where(y1 < V, ry, 0.0)
    wz1 = jnp.where(z1 < V, rz, 0.0)
    wx0 = 1.0 - rx
    wy0 = 1.0 - ry
    wz0 = 1.0 - rz
    wz0 = jnp.where(pid_ok, wz0, 0.0)
    wz1 = jnp.where(pid_ok, wz1, 0.0)

    y0s = y0 * V
    y1s = y1 * V
    z0s = z0 * (V * V)
    z1s = z1 * (V * V)
    ixy = (x0 + y0s, x1 + y0s, x0 + y1s, x1 + y1s)
    wxy = (wx0 * wy0, wx1 * wy0, wx0 * wy1, wx1 * wy1)

    out = []
    for k in range(8):
        dz, kxy = k // 4, k % 4
        idx = ixy[kxy] + (z1s if dz else z0s)
        w = wxy[kxy] * (wz1 if dz else wz0)
        inr = (idx >= lo) & (idx < lo + nrows)
        lidx = jnp.where(inr, idx - lo, 0)
        wm = jnp.where(inr, w, 0.0)
        out.append((lidx, wm))
    return out


def _sc_body(pos_hbm, amp_hbm, out_hbm, acc, pos_v, amp_v, data_v, idx_v,
             zero_v, sem, sbuf, tbuf, fsem):
    c = lax.axis_index("c")
    s = lax.axis_index("s")
    chunk0 = s * PT

    iota = lax.iota(jnp.int32, 16)
    fzero = jnp.zeros((16,), jnp.float32)
    izero = jnp.zeros((16,), jnp.int32)
    col_of = [jnp.full((16,), cc, jnp.int32) for cc in range(C)]

    def _z(i, _):
        zero_v[i, :] = fzero
        return 0
    lax.fori_loop(0, ZCHUNK, _z, 0)

    def _round(r, _):
        b = 2 * c + (r // 4)
        lo = (r % 4) * ACC_ROWS
        nrows = ACC_ROWS
        share = SHARE
        srow = s * SHARE

        def _zero(q, _):
            pltpu.sync_copy(zero_v, acc.at[pl.ds(srow + q * ZCHUNK, ZCHUNK)])
            return 0
        lax.fori_loop(0, share // ZCHUNK, _zero, 0)
        plsc.subcore_barrier()

        def _fire(f):
            q = f & (RING_CHUNKS - 1)

            @pl.when(f >= 2)
            def _recycle():
                qo = (f - 2) & (RING_CHUNKS - 1)
                pltpu.make_async_copy(
                    data_v.at[pl.ds(qo * 128, 128)], acc.at[idx_v.at[qo]],
                    sem.at[qo]).wait()

            pltpu.async_copy(
                data_v.at[pl.ds(q * 128, 128)], acc.at[idx_v.at[q]],
                sem.at[q], add=True)

        def _sub(sub, base):
            pltpu.sync_copy(
                amp_hbm.at[b, :, pl.ds(chunk0 + sub * SUB, SUB)], amp_v)
            pltpu.sync_copy(
                pos_hbm.at[b, :, pl.ds(chunk0 + sub * SUB, SUB)], pos_v)

            def _block(j, base):
                o = sub * SUB + j * 16
                px = pos_v[0, pl.ds(j * 16, 16)]
                py = pos_v[1, pl.ds(j * 16, 16)]
                pz = pos_v[2, pl.ds(j * 16, 16)]
                pid_ok = (chunk0 + o + iota) < N
                corners = _corner_terms(px, py, pz, pid_ok, lo, nrows)

                ampcols = [amp_v[cc, pl.ds(j * 16, 16)] for cc in range(C)]

                masks, cums, offs = [], [], [base]
                for k in range(8):
                    mk = corners[k][1] > 0.0
                    mi = mk.astype(jnp.int32)
                    masks.append(mk)
                    cums.append(plsc.cumsum(mi))
                    offs.append(offs[k] + jnp.sum(mi))

                for k in range(8):
                    lidx, wm = corners[k]
                    gpos = offs[k] + (cums[k] - 1)
                    rowp = gpos & (RING_ROWS - 1)
                    ck = (gpos >> 7) & (RING_CHUNKS - 1)
                    lane = gpos & 127
                    plsc.store_scatter(idx_v, (ck, lane), lidx,
                                       mask=masks[k])
                    for cc in range(C):
                        plsc.store_scatter(
                            data_v, (rowp, col_of[cc]), wm * ampcols[cc],
                            mask=masks[k])

                new_base = offs[8]

                @pl.when((new_base >> 7) > (base >> 7))
                def _maybe_fire():
                    _fire(base >> 7)

                return new_base

            return lax.fori_loop(0, BLOCKS_PER_SUB, _block, base)

        base = lax.fori_loop(0, NSUB, _sub, 0)

        cbase = base & ~jnp.int32(127)
        for g in range(8):
            gpos = cbase + g * 16 + iota
            mp = gpos >= base
            ck = (gpos >> 7) & (RING_CHUNKS - 1)
            lane = gpos & 127
            rowp = gpos & (RING_ROWS - 1)
            plsc.store_scatter(idx_v, (ck, lane), izero, mask=mp)
            for cc in range(C):
                plsc.store_scatter(data_v, (rowp, col_of[cc]), fzero,
                                   mask=mp)
        fin = base >> 7
        _fire(fin)

        @pl.when(fin >= 1)
        def _drain1():
            qo = (fin - 1) & (RING_CHUNKS - 1)
            pltpu.make_async_copy(
                data_v.at[pl.ds(qo * 128, 128)], acc.at[idx_v.at[qo]],
                sem.at[qo]).wait()

        qf = fin & (RING_CHUNKS - 1)
        pltpu.make_async_copy(
            data_v.at[pl.ds(qf * 128, 128)], acc.at[idx_v.at[qf]],
            sem.at[qf]).wait()

        plsc.subcore_barrier()

        def _dst(cc, v0):
            return out_hbm.at[b, cc, v0 // (V * V), pl.ds(v0 % (V * V), FCH)]

        def _fch(q, _):
            pltpu.sync_copy(acc.at[pl.ds(srow + q * FCH, FCH)],
                            sbuf.at[:, pl.ds(0, C)])
            par = q % 2

            @pl.when(q >= 2)
            def _recycle_t():
                offp = lo + srow + (q - 2) * FCH
                for cc in range(C):
                    pltpu.make_async_copy(
                        tbuf.at[par, cc], _dst(cc, offp),
                        fsem.at[par]).wait()

            for cc in range(C):
                for g in range(FCH // 16):
                    vec = plsc.load_gather(
                        sbuf, (iota + g * 16, col_of[cc]))
                    tbuf[par, cc, pl.ds(g * 16, 16)] = vec
            off = lo + srow + q * FCH
            for cc in range(C):
                pltpu.async_copy(
                    tbuf.at[par, cc], _dst(cc, off), fsem.at[par])
            return 0

        nch = share // FCH
        lax.fori_loop(0, nch, _fch, 0)
        for dq in (nch - 2, nch - 1):
            offp = lo + srow + dq * FCH
            for cc in range(C):
                pltpu.make_async_copy(
                    tbuf.at[dq % 2, cc], _dst(cc, offp),
                    fsem.at[dq % 2]).wait()
        plsc.subcore_barrier()
        return 0

    lax.fori_loop(0, 8, _round, 0)


def _splat_sc(pos_t, amp_p):
    kern = pl.kernel(
        _sc_body,
        out_type=jax.ShapeDtypeStruct((B, C, V, V * V), jnp.float32),
        mesh=plsc.VectorSubcoreMesh(core_axis_name="c", subcore_axis_name="s"),
        scratch_types=[
            pltpu.VMEM_SHARED((ACC_ROWS, C), jnp.float32),
            pltpu.VMEM((3, SUB), jnp.float32),
            pltpu.VMEM((C, SUB), jnp.float32),
            pltpu.VMEM((RING_ROWS, C), jnp.float32),
            pltpu.VMEM((RING_CHUNKS, 128), jnp.int32),
            pltpu.VMEM((ZCHUNK, C), jnp.float32),
            pltpu.SemaphoreType.DMA((RING_CHUNKS,)),
            pltpu.VMEM((FCH, C + 1), jnp.float32),
            pltpu.VMEM((2, C, FCH), jnp.float32),
            pltpu.SemaphoreType.DMA((2,)),
        ],
        compiler_params=pltpu.CompilerParams(
            use_tc_tiling_on_sc=False, needs_layout_passes=False),
    )
    return kern(pos_t, amp_p)


def kernel(positions, amplitudes):
    pos_t = jnp.transpose(positions, (0, 2, 1))
    pos_t = jnp.pad(pos_t, ((0, 0), (0, 0), (0, NP - N)))
    amp_p = jnp.pad(amplitudes, ((0, 0), (0, 0), (0, NP - N)))
    vol = _splat_sc(pos_t, amp_p)
    return vol.reshape(B, C, V, V, V)


# speedup vs baseline: 1.0612x; 1.0612x over previous
import functools

import jax
import jax.numpy as jnp
from jax import lax
from jax.experimental import pallas as pl
from jax.experimental.pallas import tpu as pltpu
from jax.experimental.pallas import tpu_sc as plsc

B = 4
C = 16
V = 64
M = V * V * V
N = 100000

NUM_TILES = 16
PT = 6272
NP = NUM_TILES * PT
SUB = 448
NSUB = PT // SUB
BLOCKS_PER_SUB = SUB // 16

RING_CHUNKS = 4
RING_ROWS = RING_CHUNKS * 128
FCH = 256

Z_LO = (0, 21, 42)
Z_NROWS = (21 * V * V, 21 * V * V, 22 * V * V)
ACC_ROWS = max(Z_NROWS)

ZCHUNK = 128


def _corner_terms(px, py, pz, pid_ok, lo, nrows):
    fx = (px + 0.5) * V
    fy = (py + 0.5) * V
    fz = (pz + 0.5) * V
    x0 = fx.astype(jnp.int32)
    y0 = fy.astype(jnp.int32)
    z0 = fz.astype(jnp.int32)
    rx = fx - x0.astype(jnp.float32)
    ry = fy - y0.astype(jnp.float32)
    rz = fz - z0.astype(jnp.float32)
    x1 = x0 + 1
    y1 = y0 + 1
    z1 = z0 + 1
    wx1 = jnp.where(x1 < V, rx, 0.0)
    wy1 = jnp.where(y1 < V, ry, 0.0)
    wz1 = jnp.where(z1 < V, rz, 0.0)
    wx0 = 1.0 - rx
    wy0 = 1.0 - ry
    wz0 = 1.0 - rz
    wz0 = jnp.where(pid_ok, wz0, 0.0)
    wz1 = jnp.where(pid_ok, wz1, 0.0)

    y0s = y0 * V
    y1s = y1 * V
    z0s = z0 * (V * V)
    z1s = z1 * (V * V)
    ixy = (x0 + y0s, x1 + y0s, x0 + y1s, x1 + y1s)
    wxy = (wx0 * wy0, wx1 * wy0, wx0 * wy1, wx1 * wy1)

    out = []
    for k in range(8):
        dz, kxy = k // 4, k % 4
        idx = ixy[kxy] + (z1s if dz else z0s)
        w = wxy[kxy] * (wz1 if dz else wz0)
        inr = (idx >= lo) & (idx < lo + nrows)
        lidx = jnp.where(inr, idx - lo, 0)
        wm = jnp.where(inr, w, 0.0)
        out.append((lidx, wm))
    return out


def _sc_body(pos_hbm, amp_hbm, out_hbm, acc, pos_v, amp_v, data_v, idx_v,
             zero_v, sem, sbuf, tbuf, fsem):
    c = lax.axis_index("c")
    s = lax.axis_index("s")
    chunk0 = s * PT

    iota = lax.iota(jnp.int32, 16)
    fzero = jnp.zeros((16,), jnp.float32)
    izero = jnp.zeros((16,), jnp.int32)
    col_of = [jnp.full((16,), cc, jnp.int32) for cc in range(C)]

    def _z(i, _):
        zero_v[i, :] = fzero
        return 0
    lax.fori_loop(0, ZCHUNK, _z, 0)

    for r in range(6):
        b = 2 * c + (r // 3)
        t = r % 3
        lo = Z_LO[t] * V * V
        nrows = Z_NROWS[t]
        share = nrows // NUM_TILES
        srow = s * share

        def _zero(q, _):
            pltpu.sync_copy(zero_v, acc.at[pl.ds(srow + q * ZCHUNK, ZCHUNK)])
            return 0
        lax.fori_loop(0, share // ZCHUNK, _zero, 0)
        plsc.subcore_barrier()

        def _fire(f):
            q = f & (RING_CHUNKS - 1)

            @pl.when(f >= 2)
            def _recycle():
                qo = (f - 2) & (RING_CHUNKS - 1)
                pltpu.make_async_copy(
                    data_v.at[pl.ds(qo * 128, 128)], acc.at[idx_v.at[qo]],
                    sem.at[qo]).wait()

            pltpu.async_copy(
                data_v.at[pl.ds(q * 128, 128)], acc.at[idx_v.at[q]],
                sem.at[q], add=True)

        def _sub(sub, base):
            pltpu.sync_copy(
                amp_hbm.at[b, :, pl.ds(chunk0 + sub * SUB, SUB)], amp_v)
            pltpu.sync_copy(
                pos_hbm.at[b, :, pl.ds(chunk0 + sub * SUB, SUB)], pos_v)

            def _block(j, base):
                o = sub * SUB + j * 16
                px = pos_v[0, pl.ds(j * 16, 16)]
                py = pos_v[1, pl.ds(j * 16, 16)]
                pz = pos_v[2, pl.ds(j * 16, 16)]
                pid_ok = (chunk0 + o + iota) < N
                corners = _corner_terms(px, py, pz, pid_ok, lo, nrows)

                ampcols = [amp_v[cc, pl.ds(j * 16, 16)] for cc in range(C)]

                masks, cums, offs = [], [], [base]
                for k in range(8):
                    mk = corners[k][1] > 0.0
                    mi = mk.astype(jnp.int32)
                    masks.append(mk)
                    cums.append(plsc.cumsum(mi))
                    offs.append(offs[k] + jnp.sum(mi))

                for k in range(8):
                    lidx, wm = corners[k]
                    gpos = offs[k] + (cums[k] - 1)
                    rowp = gpos & (RING_ROWS - 1)
                    ck = (gpos >> 7) & (RING_CHUNKS - 1)
                    lane = gpos & 127
                    plsc.store_scatter(idx_v, (ck, lane), lidx,
                                       mask=masks[k])
                    for cc in range(C):
                        plsc.store_scatter(
                            data_v, (rowp, col_of[cc]), wm * ampcols[cc],
                            mask=masks[k])

                new_base = offs[8]

                @pl.when((new_base >> 7) > (base >> 7))
                def _maybe_fire():
                    _fire(base >> 7)

                return new_base

            return lax.fori_loop(0, BLOCKS_PER_SUB, _block, base)

        base = lax.fori_loop(0, NSUB, _sub, 0)

        cbase = base & ~jnp.int32(127)
        for g in range(8):
            gpos = cbase + g * 16 + iota
            mp = gpos >= base
            ck = (gpos >> 7) & (RING_CHUNKS - 1)
            lane = gpos & 127
            rowp = gpos & (RING_ROWS - 1)
            plsc.store_scatter(idx_v, (ck, lane), izero, mask=mp)
            for cc in range(C):
                plsc.store_scatter(data_v, (rowp, col_of[cc]), fzero,
                                   mask=mp)
        fin = base >> 7
        _fire(fin)

        @pl.when(fin >= 1)
        def _drain1():
            qo = (fin - 1) & (RING_CHUNKS - 1)
            pltpu.make_async_copy(
                data_v.at[pl.ds(qo * 128, 128)], acc.at[idx_v.at[qo]],
                sem.at[qo]).wait()

        qf = fin & (RING_CHUNKS - 1)
        pltpu.make_async_copy(
            data_v.at[pl.ds(qf * 128, 128)], acc.at[idx_v.at[qf]],
            sem.at[qf]).wait()

        plsc.subcore_barrier()

        def _dst(cc, v0):
            return out_hbm.at[b, cc, pl.ds(v0, FCH)]

        def _fch(q, _):
            pltpu.sync_copy(acc.at[pl.ds(srow + q * FCH, FCH)], sbuf)
            par = q % 2

            @pl.when(q >= 2)
            def _recycle_t():
                offp = lo + srow + (q - 2) * FCH
                for cc in range(C):
                    pltpu.make_async_copy(
                        tbuf.at[par, cc], _dst(cc, offp),
                        fsem.at[par]).wait()

            for cc in range(C):
                for g in range(FCH // 16):
                    vec = plsc.load_gather(
                        sbuf, (iota + g * 16, col_of[cc]))
                    tbuf[par, cc, pl.ds(g * 16, 16)] = vec
            off = lo + srow + q * FCH
            for cc in range(C):
                pltpu.async_copy(
                    tbuf.at[par, cc], _dst(cc, off), fsem.at[par])
            return 0

        nch = share // FCH
        lax.fori_loop(0, nch, _fch, 0)
        for dq in (nch - 2, nch - 1):
            offp = lo + srow + dq * FCH
            for cc in range(C):
                pltpu.make_async_copy(
                    tbuf.at[dq % 2, cc], _dst(cc, offp),
                    fsem.at[dq % 2]).wait()
        plsc.subcore_barrier()


def _splat_sc(pos_t, amp_p):
    kern = pl.kernel(
        _sc_body,
        out_type=jax.ShapeDtypeStruct((B, C, M), jnp.float32),
        mesh=plsc.VectorSubcoreMesh(core_axis_name="c", subcore_axis_name="s"),
        scratch_types=[
            pltpu.VMEM_SHARED((ACC_ROWS, C), jnp.float32),
            pltpu.VMEM((3, SUB), jnp.float32),
            pltpu.VMEM((C, SUB), jnp.float32),
            pltpu.VMEM((RING_ROWS, C), jnp.float32),
            pltpu.VMEM((RING_CHUNKS, 128), jnp.int32),
            pltpu.VMEM((ZCHUNK, C), jnp.float32),
            pltpu.SemaphoreType.DMA((RING_CHUNKS,)),
            pltpu.VMEM((FCH, C), jnp.float32),
            pltpu.VMEM((2, C, FCH), jnp.float32),
            pltpu.SemaphoreType.DMA((2,)),
        ],
        compiler_params=pltpu.CompilerParams(
            use_tc_tiling_on_sc=False, needs_layout_passes=False),
    )
    return kern(pos_t, amp_p)


def kernel(positions, amplitudes):
    pos_t = jnp.transpose(positions, (0, 2, 1))
    pos_t = jnp.pad(pos_t, ((0, 0), (0, 0), (0, NP - N)))
    amp_p = jnp.pad(amplitudes, ((0, 0), (0, 0), (0, NP - N)))
    vol = _splat_sc(pos_t, amp_p)
    return vol.reshape(B, C, V, V, V)
